# baseline (speedup 1.0000x reference)
import functools

import jax
import jax.numpy as jnp
from jax import lax
from jax.experimental import pallas as pl
from jax.experimental.pallas import tpu as pltpu
from jax.experimental.pallas import tpu_sc as plsc

BATCH = 16384
NC = 2
NS = 16
NW = NC * NS
RPW = BATCH // NW
CHUNK = 128
NCH = RPW // CHUNK
LANES = 16
GW = 512
GPW = 61
COLS_PW = GPW * GW
TAIL0 = 1953 * GW
NV = 1000000
HITCAP = BATCH + LANES

_mesh = plsc.VectorSubcoreMesh(
    core_axis_name="c", subcore_axis_name="s", num_cores=NC, num_subcores=NS
)


@functools.partial(
    pl.kernel,
    out_type=(
        jax.ShapeDtypeStruct((BATCH + LANES, 128), jnp.float32),
        jax.ShapeDtypeStruct((BATCH + LANES, 128), jnp.float32),
    ),
    mesh=_mesh,
    scratch_types=(
        pltpu.VMEM((BATCH,), jnp.int32),
        pltpu.VMEM((HITCAP,), jnp.int32),
        pltpu.VMEM((HITCAP,), jnp.int32),
        pltpu.VMEM((HITCAP,), jnp.int32),
        pltpu.VMEM((HITCAP,), jnp.int32),
        pltpu.VMEM((LANES,), jnp.int32),
        (pltpu.VMEM((LANES, GW), jnp.float32),) * 2,
        pltpu.VMEM((LANES, 128), jnp.float32),
        pltpu.VMEM((LANES,), jnp.int32),
        pltpu.SemaphoreType.DMA,
        pltpu.SemaphoreType.DMA,
        pltpu.SemaphoreType.DMA,
    ),
    compiler_params=pltpu.CompilerParams(needs_layout_passes=False),
)
def _extract(
    ueT_hbm, meT_hbm, uidx_hbm, midx_hbm,
    ug_hbm, mg_hbm,
    idx_v, hitr_v, hitp_v, slr_v, slp_v, cst_v, slabs, stv_v, sti_v,
    sem0, sem1, semsc,
):
    wid = lax.axis_index("s") * NC + lax.axis_index("c")
    lo = wid * COLS_PW
    hi = jnp.where(wid == NW - 1, NV, lo + COLS_PW)
    rows = lax.iota(jnp.int32, LANES)
    sems = (sem0, sem1)

    def splat(x):
        return jnp.full((LANES,), x, jnp.int32)

    def scan_hits(n0):
        def chunk(t, cur):
            sel = splat(t * LANES) + rows
            rv = plsc.load_gather(idx_v, [sel])
            m = (rv >= lo) & (rv < hi)
            cnt = plsc.all_reduce_population_count(m)[0]
            plsc.store_compressed(cst_v.at[:], rv, mask=m)
            plsc.store_scatter(hitr_v, [splat(cur) + rows], cst_v[...])
            plsc.store_compressed(cst_v.at[:], sel, mask=m)
            plsc.store_scatter(hitp_v, [splat(cur) + rows], cst_v[...])
            return cur + cnt

        n = lax.fori_loop(0, BATCH // LANES, chunk, n0)
        plsc.store_scatter(hitr_v, [splat(n) + rows], splat(jnp.int32(1 << 30)))
        return n

    def make_process_slab(gdst_hbm):
        def process_slab(bank, col0, width, n, kk0):
            def compact(t, cur):
                sel = splat(t * LANES) + rows
                rv = plsc.load_gather(hitr_v, [sel])
                pv = plsc.load_gather(hitp_v, [sel])
                m = (rv >= col0) & (rv < col0 + width)
                cnt = plsc.all_reduce_population_count(m)[0]
                plsc.store_compressed(cst_v.at[:], rv, mask=m)
                plsc.store_scatter(slr_v, [splat(cur) + rows], cst_v[...])
                plsc.store_compressed(cst_v.at[:], pv, mask=m)
                plsc.store_scatter(slp_v, [splat(cur) + rows], cst_v[...])
                return cur + cnt

            nloc = lax.fori_loop(0, (n + LANES - 1) // LANES, compact, 0)

            def extract(k, kk):
                r = plsc.load_gather(slr_v, [splat(k)])[0]
                pos = plsc.load_gather(slp_v, [splat(k)])[0]
                col = plsc.load_gather(slabs[bank], [rows, splat(r - col0)])
                plsc.store_scatter(stv_v, [splat(kk), rows], col)
                plsc.store_scatter(
                    sti_v, [splat(kk)], splat(pos), mask=(rows == 0)
                )

                @pl.when(kk == LANES - 1)
                def _():
                    pltpu.async_copy(stv_v, gdst_hbm.at[sti_v], semsc).wait()

                return jnp.where(kk == LANES - 1, 0, kk + 1)

            return lax.fori_loop(0, nloc, extract, kk0)

        return process_slab

    def run_table(tbl_hbm, idxsrc_hbm, gdst_hbm):
        process_slab = make_process_slab(gdst_hbm)
        pltpu.sync_copy(idxsrc_hbm, idx_v)
        n = scan_hits(0)

        c0 = pl.multiple_of(lo, 128)
        pltpu.async_copy(tbl_hbm.at[:, pl.ds(c0, GW)], slabs[0], sems[0])
        c1 = pl.multiple_of(lo + GW, 128)
        pltpu.async_copy(tbl_hbm.at[:, pl.ds(c1, GW)], slabs[1], sems[1])

        gcnt = GPW + jnp.where(wid == NW - 1, 1, 0)

        def super_body(h, kk):
            for b in range(2):
                g = h * 2 + b

                @pl.when(g < gcnt)
                def _():
                    pltpu.make_async_copy(
                        tbl_hbm.at[:, pl.ds(0, GW)], slabs[b], sems[b]
                    ).wait()

                col0 = lo + g * GW
                kk = lax.cond(
                    g < gcnt,
                    lambda c: process_slab(b, col0, GW, n, c),
                    lambda c: c,
                    kk,
                )

                @pl.when(g + 2 < gcnt)
                def _():
                    cn = pl.multiple_of(lo + (g + 2) * GW, 128)
                    pltpu.async_copy(tbl_hbm.at[:, pl.ds(cn, GW)], slabs[b], sems[b])

            return kk

        nsup = (GPW + 2) // 2
        kk = lax.fori_loop(0, nsup, super_body, 0)

        @pl.when(wid == NW - 1)
        def _():
            pltpu.async_copy(
                tbl_hbm.at[:, pl.ds(pl.multiple_of(TAIL0, 128), 128)],
                slabs[0].at[:, pl.ds(0, 128)],
                sems[0],
            ).wait()

        kk = lax.cond(
            wid == NW - 1,
            lambda c: process_slab(0, TAIL0, 128, n, c),
            lambda c: c,
            kk,
        )

        @pl.when(kk > 0)
        def _():
            def padrow(k, _):
                plsc.store_scatter(
                    sti_v, [splat(k)], splat(BATCH + k), mask=(rows == 0)
                )
                return 0

            lax.fori_loop(kk, LANES, padrow, 0)
            pltpu.async_copy(stv_v, gdst_hbm.at[sti_v], semsc).wait()

    run_table(ueT_hbm, uidx_hbm, ug_hbm)
    run_table(meT_hbm, midx_hbm, mg_hbm)


@functools.partial(
    pl.kernel,
    out_type=(
        jax.ShapeDtypeStruct((NW * 128,), jnp.float32),
        jax.ShapeDtypeStruct((BATCH,), jnp.float32),
    ),
    mesh=_mesh,
    scratch_types=(
        pltpu.VMEM((NCH, CHUNK), jnp.int32),
        pltpu.VMEM((NCH, CHUNK), jnp.int32),
        pltpu.VMEM((RPW,), jnp.float32),
        pltpu.VMEM((RPW,), jnp.float32),
        pltpu.VMEM((CHUNK, 128), jnp.float32),
        pltpu.VMEM((CHUNK, 128), jnp.float32),
        pltpu.VMEM((128,), jnp.float32),
        pltpu.VMEM((RPW,), jnp.float32),
        pltpu.SemaphoreType.DMA,
    ),
    compiler_params=pltpu.CompilerParams(
        use_tc_tiling_on_sc=False, needs_layout_passes=False
    ),
)
def _dot_bias(
    ug_hbm, mg_hbm, uidx_hbm, midx_hbm, ub_hbm, mb_hbm,
    partials_hbm, bsum_hbm,
    uidx_v, midx_v, ub_v, mb_v, us_v, ms_v, stage_v, bs_v, sem,
):
    wid = lax.axis_index("s") * NC + lax.axis_index("c")
    rows = lax.iota(jnp.int32, LANES)

    idx_cps = []
    for c in range(NCH):
        idx_cps.append(pltpu.async_copy(uidx_hbm.at[wid * NCH + c], uidx_v.at[c], sem))
        idx_cps.append(pltpu.async_copy(midx_hbm.at[wid * NCH + c], midx_v.at[c], sem))
    for cp in idx_cps:
        cp.wait()

    cps = []
    for c in range(NCH):
        sl = pl.ds(c * CHUNK, CHUNK)
        cps.append(pltpu.async_copy(ub_hbm.at[uidx_v.at[c]], ub_v.at[sl], sem))
        cps.append(pltpu.async_copy(mb_hbm.at[midx_v.at[c]], mb_v.at[sl], sem))
    for cp in cps:
        cp.wait()

    acc = jnp.zeros((LANES,), jnp.float32)
    for c in range(NCH):
        base = wid * RPW + c * CHUNK
        cpu = pltpu.async_copy(ug_hbm.at[pl.ds(base, CHUNK)], us_v, sem)
        cpm = pltpu.async_copy(mg_hbm.at[pl.ds(base, CHUNK)], ms_v, sem)
        cpu.wait()
        cpm.wait()

        def dot_body(j, a):
            sl = pl.ds(0, LANES)
            return a + us_v[j, sl] * ms_v[j, sl]

        acc = lax.fori_loop(0, CHUNK, dot_body, acc, unroll=8)

    plsc.store_scatter(stage_v, [rows], acc)
    pltpu.sync_copy(stage_v, partials_hbm.at[pl.ds(wid * 128, 128)])

    def bias_body(k, carry):
        sl = pl.ds(k * LANES, LANES)
        bs_v[sl] = ub_v[sl] + mb_v[sl]
        return carry

    lax.fori_loop(0, RPW // LANES, bias_body, 0, unroll=4)
    pltpu.sync_copy(bs_v, bsum_hbm.at[pl.ds(wid * RPW, RPW)])


@functools.partial(
    pl.kernel,
    out_type=jax.ShapeDtypeStruct((BATCH,), jnp.float32),
    mesh=_mesh,
    scratch_types=(
        pltpu.VMEM((NW * 128,), jnp.float32),
        pltpu.VMEM((RPW,), jnp.float32),
        pltpu.VMEM((RPW,), jnp.float32),
    ),
    compiler_params=pltpu.CompilerParams(
        use_tc_tiling_on_sc=False, needs_layout_passes=False
    ),
)
def _sigmoid_fin(partials_hbm, bsum_hbm, out_hbm, part_v, b_v, o_v):
    wid = lax.axis_index("s") * NC + lax.axis_index("c")
    pltpu.sync_copy(partials_hbm, part_v)
    pltpu.sync_copy(bsum_hbm.at[pl.ds(wid * RPW, RPW)], b_v)

    acc = part_v[pl.ds(0, LANES)]
    for w in range(1, NW):
        acc = acc + part_v[pl.ds(w * 128, LANES)]
    s = jnp.sum(acc)

    def sig_body(k, carry):
        sl = pl.ds(k * LANES, LANES)
        x = s + b_v[sl]
        o_v[sl] = 1.0 / (1.0 + jnp.exp(-x))
        return carry

    lax.fori_loop(0, RPW // LANES, sig_body, 0, unroll=4)
    pltpu.sync_copy(o_v, out_hbm.at[pl.ds(wid * RPW, RPW)])


def kernel(inputs, user_embedding, movie_embedding, user_bias, movie_bias):
    uidx = inputs[:, 0]
    midx = inputs[:, 1]
    uidx2 = uidx.reshape(NW * NCH, CHUNK)
    midx2 = midx.reshape(NW * NCH, CHUNK)
    ub = user_bias.reshape(-1)
    mb = movie_bias.reshape(-1)
    ug, mg = _extract(user_embedding.T, movie_embedding.T, uidx, midx)
    partials, bsum = _dot_bias(ug, mg, uidx2, midx2, ub, mb)
    out = _sigmoid_fin(partials, bsum)
    return out.reshape(BATCH, 1)
